# Initial kernel scaffold; baseline (speedup 1.0000x reference)
import functools

import jax
import jax.numpy as jnp
import numpy as np
from jax import lax
from jax.experimental import pallas as pl
from jax.experimental.pallas import tpu as pltpu
from jax.experimental.pallas import tpu_sc as plsc

N_NODES = 100000
N_EDGES = 6400000
NUM_TYPES = 16
FACTOR = 1.0 / np.sqrt(64.0)

NC = 2
NS = 16
NW = NC * NS
L = 16

EPW = N_EDGES // NW
K = 2000
CHUNKS = EPW // K
GROUPS = K // L

NSEG = 6256
NPAD = NS * NSEG


def _sc_body(energy_hbm, center_hbm, neigh_hbm, species_hbm, scale_hbm,
             out_hbm, species_v, scale_v, e_v, c_v, n_v, vals_v, zbuf,
             accum_sh):
    cid = lax.axis_index("c")
    sid = lax.axis_index("s")
    wid = cid * NS + sid

    pltpu.sync_copy(species_hbm, species_v)
    pltpu.sync_copy(scale_hbm, scale_v)

    def zbody(i, _):
        zbuf[pl.ds(i * L, L)] = jnp.zeros((L,), jnp.float32)
        return _

    lax.fori_loop(0, NSEG // L, zbody, None)
    pltpu.sync_copy(zbuf, accum_sh.at[pl.ds(sid * NSEG, NSEG)])
    plsc.subcore_barrier()

    def chunk_body(t, _):
        base = wid * EPW + t * K
        pltpu.sync_copy(energy_hbm.at[pl.ds(base, K)], e_v)
        pltpu.sync_copy(center_hbm.at[pl.ds(base, K)], c_v)
        pltpu.sync_copy(neigh_hbm.at[pl.ds(base, K)], n_v)

        def gbody(g, _):
            off = g * L
            ci = c_v[pl.ds(off, L)]
            ni = n_v[pl.ds(off, L)]
            sc = plsc.load_gather(species_v, [ci])
            sn = plsc.load_gather(species_v, [ni])
            comb = sc * NUM_TYPES + sn
            sv = plsc.load_gather(scale_v, [comb])
            vals_v[pl.ds(off, L)] = e_v[pl.ds(off, L)] * sv
            return _

        lax.fori_loop(0, GROUPS, gbody, None)
        pltpu.sync_copy(vals_v, accum_sh.at[c_v], add=True)
        return _

    lax.fori_loop(0, CHUNKS, chunk_body, None)
    plsc.subcore_barrier()

    pltpu.sync_copy(accum_sh.at[pl.ds(sid * NSEG, NSEG)], zbuf)
    pltpu.sync_copy(zbuf, out_hbm.at[cid, pl.ds(sid * NSEG, NSEG)])


@jax.jit
def _sc_call(energy, center, neigh, species, scale):
    mesh = plsc.VectorSubcoreMesh(core_axis_name="c", subcore_axis_name="s")
    return pl.kernel(
        _sc_body,
        out_type=jax.ShapeDtypeStruct((NC, NPAD), jnp.float32),
        mesh=mesh,
        scratch_types=[
            pltpu.VMEM((N_NODES,), jnp.int32),
            pltpu.VMEM((NUM_TYPES * NUM_TYPES,), jnp.float32),
            pltpu.VMEM((K,), jnp.float32),
            pltpu.VMEM((K,), jnp.int32),
            pltpu.VMEM((K,), jnp.int32),
            pltpu.VMEM((K,), jnp.float32),
            pltpu.VMEM((NSEG,), jnp.float32),
            pltpu.VMEM_SHARED((NPAD,), jnp.float32),
        ],
    )(energy, center, neigh, species, scale)


def kernel(edge_energy, per_edge_scales, edge_index, atom_types):
    energy = edge_energy.reshape(N_EDGES)
    center = edge_index[0]
    neigh = edge_index[1]
    species = atom_types.reshape(N_NODES)
    scale = (per_edge_scales * FACTOR).reshape(NUM_TYPES * NUM_TYPES)
    partials = _sc_call(energy, center, neigh, species, scale)
    return (partials[0, :N_NODES] + partials[1, :N_NODES])[:, None]


# speedup vs baseline: 384.3226x; 384.3226x over previous
import functools

import jax
import jax.numpy as jnp
import numpy as np
from jax import lax
from jax.experimental import pallas as pl
from jax.experimental.pallas import tpu as pltpu
from jax.experimental.pallas import tpu_sc as plsc

N_NODES = 100000
N_EDGES = 6400000
NUM_TYPES = 16
FACTOR = 1.0 / np.sqrt(64.0)

NC = 2
NS = 16
NW = NC * NS
L = 16

EPW = N_EDGES // NW
K = 2000
CHUNKS = EPW // K
GROUPS = K // L

NSEG = 6256
NPAD = NS * NSEG


def _sc_body(energy_hbm, center_hbm, neigh_hbm, species_hbm, scale_hbm,
             out_hbm, species_v, scale_v, e_v, c_v, n_v, vals_v, zbuf,
             accum_sh):
    cid = lax.axis_index("c")
    sid = lax.axis_index("s")
    wid = cid * NS + sid

    pltpu.sync_copy(species_hbm, species_v)
    pltpu.sync_copy(scale_hbm, scale_v)

    def zbody(i, _):
        zbuf[pl.ds(i * L, L)] = jnp.zeros((L,), jnp.float32)
        return _

    lax.fori_loop(0, NSEG // L, zbody, None)
    pltpu.sync_copy(zbuf, accum_sh.at[pl.ds(sid * NSEG, NSEG)])
    plsc.subcore_barrier()

    def chunk_body(t, _):
        base = wid * EPW + t * K
        pltpu.sync_copy(energy_hbm.at[pl.ds(base, K)], e_v)
        pltpu.sync_copy(center_hbm.at[pl.ds(base, K)], c_v)
        pltpu.sync_copy(neigh_hbm.at[pl.ds(base, K)], n_v)

        def gbody(g, _):
            off = g * L
            ci = c_v[pl.ds(off, L)]
            ni = n_v[pl.ds(off, L)]
            sc = plsc.load_gather(species_v, [ci])
            sn = plsc.load_gather(species_v, [ni])
            comb = sc * NUM_TYPES + sn
            sv = plsc.load_gather(scale_v, [comb])
            vals_v[pl.ds(off, L)] = e_v[pl.ds(off, L)] * sv
            return _

        lax.fori_loop(0, GROUPS, gbody, None)
        pltpu.sync_copy(vals_v, accum_sh.at[c_v], add=True)
        return _

    lax.fori_loop(0, CHUNKS, chunk_body, None)
    plsc.subcore_barrier()

    pltpu.sync_copy(accum_sh.at[pl.ds(sid * NSEG, NSEG)], zbuf)
    pltpu.sync_copy(zbuf, out_hbm.at[pl.ds(cid * NPAD + sid * NSEG, NSEG)])


@jax.jit
def _sc_call(energy, center, neigh, species, scale):
    mesh = plsc.VectorSubcoreMesh(core_axis_name="c", subcore_axis_name="s")
    return pl.kernel(
        _sc_body,
        out_type=jax.ShapeDtypeStruct((NC * NPAD,), jnp.float32),
        mesh=mesh,
        compiler_params=pltpu.CompilerParams(needs_layout_passes=False),
        scratch_types=[
            pltpu.VMEM((N_NODES,), jnp.int32),
            pltpu.VMEM((NUM_TYPES * NUM_TYPES,), jnp.float32),
            pltpu.VMEM((K,), jnp.float32),
            pltpu.VMEM((K,), jnp.int32),
            pltpu.VMEM((K,), jnp.int32),
            pltpu.VMEM((K,), jnp.float32),
            pltpu.VMEM((NSEG,), jnp.float32),
            pltpu.VMEM_SHARED((NPAD,), jnp.float32),
        ],
    )(energy, center, neigh, species, scale)


def kernel(edge_energy, per_edge_scales, edge_index, atom_types):
    energy = edge_energy.reshape(N_EDGES)
    center = edge_index[0]
    neigh = edge_index[1]
    species = atom_types.reshape(N_NODES)
    scale = (per_edge_scales * FACTOR).reshape(NUM_TYPES * NUM_TYPES)
    partials = _sc_call(energy, center, neigh, species, scale)
    return (partials[:N_NODES] + partials[NPAD:NPAD + N_NODES])[:, None]


# speedup vs baseline: 436.5551x; 1.1359x over previous
import functools

import jax
import jax.numpy as jnp
import numpy as np
from jax import lax
from jax.experimental import pallas as pl
from jax.experimental.pallas import tpu as pltpu
from jax.experimental.pallas import tpu_sc as plsc

N_NODES = 100000
N_EDGES = 6400000
NUM_TYPES = 16
FACTOR = 1.0 / np.sqrt(64.0)

NC = 2
NS = 16
NW = NC * NS
L = 16

EPW = N_EDGES // NW
K = 2000
CHUNKS = EPW // K
GROUPS = K // L

NSEG = 6256
NPAD = NS * NSEG


def _sc_body(energy_hbm, center_hbm, neigh_hbm, species_hbm, scale_hbm,
             out_hbm, species_v, scale_v, e_v, c_v, n_v, vals_v, zbuf,
             accum_sh):
    cid = lax.axis_index("c")
    sid = lax.axis_index("s")
    wid = cid * NS + sid

    pltpu.sync_copy(species_hbm, species_v)
    pltpu.sync_copy(scale_hbm, scale_v)

    def zbody(i, _):
        zbuf[pl.ds(i * L, L)] = jnp.zeros((L,), jnp.float32)
        return _

    lax.fori_loop(0, NSEG // L, zbody, None)
    pltpu.sync_copy(zbuf, accum_sh.at[pl.ds(sid * NSEG, NSEG)])
    plsc.subcore_barrier()

    def chunk_body(t, _):
        base = wid * EPW + t * K
        pltpu.sync_copy(energy_hbm.at[pl.ds(base, K)], e_v)
        pltpu.sync_copy(center_hbm.at[pl.ds(base, K)], c_v)
        pltpu.sync_copy(neigh_hbm.at[pl.ds(base, K)], n_v)

        def gbody(g, _):
            off = g * L
            ci = c_v[pl.ds(off, L)]
            ni = n_v[pl.ds(off, L)]
            sc = plsc.load_gather(species_v, [ci])
            sn = plsc.load_gather(species_v, [ni])
            comb = sc * NUM_TYPES + sn
            sv = plsc.load_gather(scale_v, [comb])
            vals_v[pl.ds(off, L)] = e_v[pl.ds(off, L)] * sv
            return _

        lax.fori_loop(0, GROUPS, gbody, None)
        return _

    lax.fori_loop(0, CHUNKS, chunk_body, None)
    plsc.subcore_barrier()

    pltpu.sync_copy(accum_sh.at[pl.ds(sid * NSEG, NSEG)], zbuf)
    pltpu.sync_copy(zbuf, out_hbm.at[pl.ds(cid * NPAD + sid * NSEG, NSEG)])


@jax.jit
def _sc_call(energy, center, neigh, species, scale):
    mesh = plsc.VectorSubcoreMesh(core_axis_name="c", subcore_axis_name="s")
    return pl.kernel(
        _sc_body,
        out_type=jax.ShapeDtypeStruct((NC * NPAD,), jnp.float32),
        mesh=mesh,
        compiler_params=pltpu.CompilerParams(needs_layout_passes=False),
        scratch_types=[
            pltpu.VMEM((N_NODES,), jnp.int32),
            pltpu.VMEM((NUM_TYPES * NUM_TYPES,), jnp.float32),
            pltpu.VMEM((K,), jnp.float32),
            pltpu.VMEM((K,), jnp.int32),
            pltpu.VMEM((K,), jnp.int32),
            pltpu.VMEM((K,), jnp.float32),
            pltpu.VMEM((NSEG,), jnp.float32),
            pltpu.VMEM_SHARED((NPAD,), jnp.float32),
        ],
    )(energy, center, neigh, species, scale)


def kernel(edge_energy, per_edge_scales, edge_index, atom_types):
    energy = edge_energy.reshape(N_EDGES)
    center = edge_index[0]
    neigh = edge_index[1]
    species = atom_types.reshape(N_NODES)
    scale = (per_edge_scales * FACTOR).reshape(NUM_TYPES * NUM_TYPES)
    partials = _sc_call(energy, center, neigh, species, scale)
    return (partials[:N_NODES] + partials[NPAD:NPAD + N_NODES])[:, None]


# speedup vs baseline: 563.0961x; 1.2899x over previous
import functools

import jax
import jax.numpy as jnp
import numpy as np
from jax import lax
from jax.experimental import pallas as pl
from jax.experimental.pallas import tpu as pltpu
from jax.experimental.pallas import tpu_sc as plsc

N_NODES = 100000
N_EDGES = 6400000
NUM_TYPES = 16
FACTOR = 1.0 / np.sqrt(64.0)

NC = 2
NS = 16
NW = NC * NS
L = 16

EPW = N_EDGES // NW
K = 2000
CHUNKS = EPW // K
GROUPS = K // L

NSEG = 6256
NPAD = NS * NSEG


def _sc_body(energy_hbm, center_hbm, neigh_hbm, species_hbm, scale_hbm,
             out_hbm, species_v, scale_v, e_v, c_v, n_v, vals_v, zbuf,
             accum_sh):
    cid = lax.axis_index("c")
    sid = lax.axis_index("s")
    wid = cid * NS + sid

    pltpu.sync_copy(species_hbm, species_v)
    pltpu.sync_copy(scale_hbm, scale_v)

    def zbody(i, _):
        zbuf[pl.ds(i * L, L)] = jnp.zeros((L,), jnp.float32)
        return _

    lax.fori_loop(0, NSEG // L, zbody, None)
    pltpu.sync_copy(zbuf, accum_sh.at[pl.ds(sid * NSEG, NSEG)])
    plsc.subcore_barrier()

    def chunk_body(t, _):
        base = wid * EPW + t * K
        pltpu.sync_copy(energy_hbm.at[pl.ds(base, K)], e_v)
        pltpu.sync_copy(center_hbm.at[pl.ds(base, K)], c_v)
        pltpu.sync_copy(neigh_hbm.at[pl.ds(base, K)], n_v)

        def gbody(g, _):
            off = g * L
            ci = c_v[pl.ds(off, L)]
            ni = n_v[pl.ds(off, L)]
            sc = plsc.load_gather(species_v, [ci])
            sn = plsc.load_gather(species_v, [ni])
            comb = sc * NUM_TYPES + sn
            sv = plsc.load_gather(scale_v, [comb])
            vals_v[pl.ds(off, L)] = e_v[pl.ds(off, L)] * sv
            return _

        lax.fori_loop(0, 1, gbody, None)
        pltpu.sync_copy(vals_v, accum_sh.at[c_v], add=True)
        return _

    lax.fori_loop(0, CHUNKS, chunk_body, None)
    plsc.subcore_barrier()

    pltpu.sync_copy(accum_sh.at[pl.ds(sid * NSEG, NSEG)], zbuf)
    pltpu.sync_copy(zbuf, out_hbm.at[pl.ds(cid * NPAD + sid * NSEG, NSEG)])


@jax.jit
def _sc_call(energy, center, neigh, species, scale):
    mesh = plsc.VectorSubcoreMesh(core_axis_name="c", subcore_axis_name="s")
    return pl.kernel(
        _sc_body,
        out_type=jax.ShapeDtypeStruct((NC * NPAD,), jnp.float32),
        mesh=mesh,
        compiler_params=pltpu.CompilerParams(needs_layout_passes=False),
        scratch_types=[
            pltpu.VMEM((N_NODES,), jnp.int32),
            pltpu.VMEM((NUM_TYPES * NUM_TYPES,), jnp.float32),
            pltpu.VMEM((K,), jnp.float32),
            pltpu.VMEM((K,), jnp.int32),
            pltpu.VMEM((K,), jnp.int32),
            pltpu.VMEM((K,), jnp.float32),
            pltpu.VMEM((NSEG,), jnp.float32),
            pltpu.VMEM_SHARED((NPAD,), jnp.float32),
        ],
    )(energy, center, neigh, species, scale)


def kernel(edge_energy, per_edge_scales, edge_index, atom_types):
    energy = edge_energy.reshape(N_EDGES)
    center = edge_index[0]
    neigh = edge_index[1]
    species = atom_types.reshape(N_NODES)
    scale = (per_edge_scales * FACTOR).reshape(NUM_TYPES * NUM_TYPES)
    partials = _sc_call(energy, center, neigh, species, scale)
    return (partials[:N_NODES] + partials[NPAD:NPAD + N_NODES])[:, None]


# speedup vs baseline: 605.8541x; 1.0759x over previous
import functools

import jax
import jax.numpy as jnp
import numpy as np
from jax import lax
from jax.experimental import pallas as pl
from jax.experimental.pallas import tpu as pltpu
from jax.experimental.pallas import tpu_sc as plsc

N_NODES = 100000
N_EDGES = 6400000
NUM_TYPES = 16
FACTOR = 1.0 / np.sqrt(64.0)

NC = 2
NS = 16
NW = NC * NS
L = 16

EPW = N_EDGES // NW
K = 2000
CHUNKS = EPW // K
GROUPS = K // L

NSEG = 6256
NPAD = NS * NSEG


def _sc_body(energy_hbm, center_hbm, neigh_hbm, species_hbm, scale_hbm,
             out_hbm, species_v, scale_v, e_v, c_v, n_v, vals_v,
             e_v2, c_v2, n_v2, vals_v2, sem0, sem1, zbuf, accum_sh):
    cid = lax.axis_index("c")
    sid = lax.axis_index("s")
    wid = cid * NS + sid

    pltpu.sync_copy(species_hbm, species_v)
    pltpu.sync_copy(scale_hbm, scale_v)

    def zbody(i, _):
        zbuf[pl.ds(i * L, L)] = jnp.zeros((L,), jnp.float32)
        return _

    lax.fori_loop(0, NSEG // L, zbody, None)
    pltpu.sync_copy(zbuf, accum_sh.at[pl.ds(sid * NSEG, NSEG)])
    plsc.subcore_barrier()

    e_b = (e_v, e_v2)
    c_b = (c_v, c_v2)
    n_b = (n_v, n_v2)
    v_b = (vals_v, vals_v2)
    sem_b = (sem0, sem1)

    def fire_in(t, b):
        base = wid * EPW + t * K
        pltpu.async_copy(energy_hbm.at[pl.ds(base, K)], e_b[b], sem_b[b])
        pltpu.async_copy(center_hbm.at[pl.ds(base, K)], c_b[b], sem_b[b])
        pltpu.async_copy(neigh_hbm.at[pl.ds(base, K)], n_b[b], sem_b[b])

    def wait_in(t, b):
        base = wid * EPW + t * K
        pltpu.make_async_copy(energy_hbm.at[pl.ds(base, K)], e_b[b],
                              sem_b[b]).wait()
        pltpu.make_async_copy(center_hbm.at[pl.ds(base, K)], c_b[b],
                              sem_b[b]).wait()
        pltpu.make_async_copy(neigh_hbm.at[pl.ds(base, K)], n_b[b],
                              sem_b[b]).wait()

    def compute_scatter(t, b):
        def gbody(g, _):
            off = g * L
            ci = c_b[b][pl.ds(off, L)]
            ni = n_b[b][pl.ds(off, L)]
            sc = plsc.load_gather(species_v, [ci])
            sn = plsc.load_gather(species_v, [ni])
            comb = sc * NUM_TYPES + sn
            v_b[b][pl.ds(off, L)] = e_b[b][pl.ds(off, L)] * \
                plsc.load_gather(scale_v, [comb])
            return _

        lax.fori_loop(0, GROUPS, gbody, None)
        pltpu.sync_copy(v_b[b], accum_sh.at[c_b[b]], add=True)

    fire_in(0, 0)

    def block_body(t2, _):
        for j in range(2):
            t = t2 * 2 + j
            wait_in(t, j)
            fire_in(t + 1, j ^ 1)
            compute_scatter(t, j)
        return _

    lax.fori_loop(0, (CHUNKS - 2) // 2, block_body, None)
    wait_in(CHUNKS - 2, 0)
    fire_in(CHUNKS - 1, 1)
    compute_scatter(CHUNKS - 2, 0)
    wait_in(CHUNKS - 1, 1)
    compute_scatter(CHUNKS - 1, 1)
    plsc.subcore_barrier()

    pltpu.sync_copy(accum_sh.at[pl.ds(sid * NSEG, NSEG)], zbuf)
    pltpu.sync_copy(zbuf, out_hbm.at[pl.ds(cid * NPAD + sid * NSEG, NSEG)])


@jax.jit
def _sc_call(energy, center, neigh, species, scale):
    mesh = plsc.VectorSubcoreMesh(core_axis_name="c", subcore_axis_name="s")
    return pl.kernel(
        _sc_body,
        out_type=jax.ShapeDtypeStruct((NC * NPAD,), jnp.float32),
        mesh=mesh,
        compiler_params=pltpu.CompilerParams(needs_layout_passes=False),
        scratch_types=[
            pltpu.VMEM((N_NODES,), jnp.int32),
            pltpu.VMEM((NUM_TYPES * NUM_TYPES,), jnp.float32),
            pltpu.VMEM((K,), jnp.float32),
            pltpu.VMEM((K,), jnp.int32),
            pltpu.VMEM((K,), jnp.int32),
            pltpu.VMEM((K,), jnp.float32),
            pltpu.VMEM((K,), jnp.float32),
            pltpu.VMEM((K,), jnp.int32),
            pltpu.VMEM((K,), jnp.int32),
            pltpu.VMEM((K,), jnp.float32),
            pltpu.SemaphoreType.DMA,
            pltpu.SemaphoreType.DMA,
            pltpu.VMEM((NSEG,), jnp.float32),
            pltpu.VMEM_SHARED((NPAD,), jnp.float32),
        ],
    )(energy, center, neigh, species, scale)


def kernel(edge_energy, per_edge_scales, edge_index, atom_types):
    energy = edge_energy.reshape(N_EDGES)
    center = edge_index[0]
    neigh = edge_index[1]
    species = atom_types.reshape(N_NODES)
    scale = (per_edge_scales * FACTOR).reshape(NUM_TYPES * NUM_TYPES)
    partials = _sc_call(energy, center, neigh, species, scale)
    return (partials[:N_NODES] + partials[NPAD:NPAD + N_NODES])[:, None]


# speedup vs baseline: 925.9641x; 1.5284x over previous
import functools

import jax
import jax.numpy as jnp
import numpy as np
from jax import lax
from jax.experimental import pallas as pl
from jax.experimental.pallas import tpu as pltpu
from jax.experimental.pallas import tpu_sc as plsc

N_NODES = 100000
N_EDGES = 6400000
NUM_TYPES = 16
FACTOR = 1.0 / np.sqrt(64.0)

NC = 2
NS = 16
NW = NC * NS
L = 16

EPW = N_EDGES // NW
K = 2000
CHUNKS = EPW // K
GROUPS = K // L

NSEG = 6256
NPAD = NS * NSEG


def _sc_body(energy_hbm, center_hbm, neigh_hbm, species_hbm, scale_hbm,
             out_hbm, species_v, scale_v, e_v, c_v, n_v, vals_v,
             e_v2, c_v2, n_v2, vals_v2, sem0, sem1, zbuf, accum_sh):
    cid = lax.axis_index("c")
    sid = lax.axis_index("s")
    wid = cid * NS + sid

    pltpu.sync_copy(species_hbm, species_v)
    pltpu.sync_copy(scale_hbm, scale_v)

    def zbody(i, _):
        zbuf[pl.ds(i * L, L)] = jnp.zeros((L,), jnp.float32)
        return _

    lax.fori_loop(0, NSEG // L, zbody, None)
    pltpu.sync_copy(zbuf, accum_sh.at[pl.ds(sid * NSEG, NSEG)])
    plsc.subcore_barrier()

    e_b = (e_v, e_v2)
    c_b = (c_v, c_v2)
    n_b = (n_v, n_v2)
    v_b = (vals_v, vals_v2)
    sem_b = (sem0, sem1)

    def fire_in(t, b):
        base = wid * EPW + t * K
        pltpu.async_copy(energy_hbm.at[pl.ds(base, K)], e_b[b], sem_b[b])
        pltpu.async_copy(center_hbm.at[pl.ds(base, K)], c_b[b], sem_b[b])
        pltpu.async_copy(neigh_hbm.at[pl.ds(base, K)], n_b[b], sem_b[b])

    def wait_in(t, b):
        base = wid * EPW + t * K
        pltpu.make_async_copy(energy_hbm.at[pl.ds(base, K)], e_b[b],
                              sem_b[b]).wait()
        pltpu.make_async_copy(center_hbm.at[pl.ds(base, K)], c_b[b],
                              sem_b[b]).wait()
        pltpu.make_async_copy(neigh_hbm.at[pl.ds(base, K)], n_b[b],
                              sem_b[b]).wait()

    def compute_scatter(t, b):
        @plsc.parallel_loop(0, K, step=L, unroll=4)
        def gbody(off):
            ci = c_b[b][pl.ds(off, L)]
            ni = n_b[b][pl.ds(off, L)]
            sc = plsc.load_gather(species_v, [ci])
            sn = plsc.load_gather(species_v, [ni])
            comb = sc * NUM_TYPES + sn
            v_b[b][pl.ds(off, L)] = e_b[b][pl.ds(off, L)] * \
                plsc.load_gather(scale_v, [comb])
        pltpu.sync_copy(v_b[b], accum_sh.at[c_b[b]], add=True)

    fire_in(0, 0)

    def block_body(t2, _):
        for j in range(2):
            t = t2 * 2 + j
            wait_in(t, j)
            fire_in(t + 1, j ^ 1)
            compute_scatter(t, j)
        return _

    lax.fori_loop(0, (CHUNKS - 2) // 2, block_body, None)
    wait_in(CHUNKS - 2, 0)
    fire_in(CHUNKS - 1, 1)
    compute_scatter(CHUNKS - 2, 0)
    wait_in(CHUNKS - 1, 1)
    compute_scatter(CHUNKS - 1, 1)
    plsc.subcore_barrier()

    pltpu.sync_copy(accum_sh.at[pl.ds(sid * NSEG, NSEG)], zbuf)
    pltpu.sync_copy(zbuf, out_hbm.at[pl.ds(cid * NPAD + sid * NSEG, NSEG)])


@jax.jit
def _sc_call(energy, center, neigh, species, scale):
    mesh = plsc.VectorSubcoreMesh(core_axis_name="c", subcore_axis_name="s")
    return pl.kernel(
        _sc_body,
        out_type=jax.ShapeDtypeStruct((NC * NPAD,), jnp.float32),
        mesh=mesh,
        compiler_params=pltpu.CompilerParams(needs_layout_passes=False),
        scratch_types=[
            pltpu.VMEM((N_NODES,), jnp.int32),
            pltpu.VMEM((NUM_TYPES * NUM_TYPES,), jnp.float32),
            pltpu.VMEM((K,), jnp.float32),
            pltpu.VMEM((K,), jnp.int32),
            pltpu.VMEM((K,), jnp.int32),
            pltpu.VMEM((K,), jnp.float32),
            pltpu.VMEM((K,), jnp.float32),
            pltpu.VMEM((K,), jnp.int32),
            pltpu.VMEM((K,), jnp.int32),
            pltpu.VMEM((K,), jnp.float32),
            pltpu.SemaphoreType.DMA,
            pltpu.SemaphoreType.DMA,
            pltpu.VMEM((NSEG,), jnp.float32),
            pltpu.VMEM_SHARED((NPAD,), jnp.float32),
        ],
    )(energy, center, neigh, species, scale)


def kernel(edge_energy, per_edge_scales, edge_index, atom_types):
    energy = edge_energy.reshape(N_EDGES)
    center = edge_index[0]
    neigh = edge_index[1]
    species = atom_types.reshape(N_NODES)
    scale = (per_edge_scales * FACTOR).reshape(NUM_TYPES * NUM_TYPES)
    partials = _sc_call(energy, center, neigh, species, scale)
    return (partials[:N_NODES] + partials[NPAD:NPAD + N_NODES])[:, None]
